# baseline (device time: 16314 ns/iter reference)
import jax
import jax.numpy as jnp
from jax import lax
from jax.experimental import pallas as pl
from jax.experimental.pallas import tpu as pltpu

C = 4


def kernel(x):
    m, n = x.shape
    q = m // 4
    qc = q // C

    def body(x_ref, out_ref, graw_ref, zsend_sems, zrecv_sems,
             f_send_sems, f_recv_sems):
        my_x = lax.axis_index("x")
        my_y = lax.axis_index("y")
        my_z = lax.axis_index("z")
        my_p = my_x * 2 + my_y

        zpartner = (my_x, my_y, 1 - my_z)
        xpeer = (1 - my_x, my_y, my_z)
        ypeer = (my_x, 1 - my_y, my_z)
        dpeer = (1 - my_x, 1 - my_y, my_z)
        px = (1 - my_x) * 2 + my_y
        py = my_x * 2 + (1 - my_y)
        pd = (1 - my_x) * 2 + (1 - my_y)

        barrier_sem = pltpu.get_barrier_semaphore()
        for peer in (zpartner, xpeer, ypeer, dpeer):
            pl.semaphore_signal(
                barrier_sem, inc=1,
                device_id=peer, device_id_type=pl.DeviceIdType.MESH,
            )
        pl.semaphore_wait(barrier_sem, 4)

        zrdmas = []
        for c in range(C):
            rows = pl.ds(my_p * q + c * qc, qc)
            r = pltpu.make_async_remote_copy(
                src_ref=x_ref.at[rows, :],
                dst_ref=graw_ref.at[rows, :],
                send_sem=zsend_sems.at[c],
                recv_sem=zrecv_sems.at[c],
                device_id=zpartner,
                device_id_type=pl.DeviceIdType.MESH,
            )
            r.start()
            zrdmas.append(r)

        fan = []
        for c in range(C):
            zrdmas[c].wait_recv()
            rows = pl.ds(my_p * q + c * qc, qc)
            for i, peer in enumerate((xpeer, ypeer, dpeer)):
                r = pltpu.make_async_remote_copy(
                    src_ref=graw_ref.at[rows, :],
                    dst_ref=graw_ref.at[rows, :],
                    send_sem=f_send_sems.at[i, c],
                    recv_sem=f_recv_sems.at[my_p, c],
                    device_id=peer,
                    device_id_type=pl.DeviceIdType.MESH,
                )
                r.start()
                fan.append(r)

        myrows = pl.ds(my_p * q, q)
        out_ref[myrows, :] = x_ref[myrows, :] + graw_ref[myrows, :]

        for sp in (px, py, pd):
            for c in range(C):
                rows = pl.ds(sp * q + c * qc, qc)
                rr = pltpu.make_async_remote_copy(
                    src_ref=graw_ref.at[rows, :],
                    dst_ref=graw_ref.at[rows, :],
                    send_sem=f_send_sems.at[0, 0],
                    recv_sem=f_recv_sems.at[sp, c],
                    device_id=xpeer,
                    device_id_type=pl.DeviceIdType.MESH,
                )
                rr.wait_recv()
            qrows = pl.ds(sp * q, q)
            out_ref[qrows, :] = x_ref[qrows, :] + graw_ref[qrows, :]

        for r in zrdmas:
            r.wait_send()
        for r in fan:
            r.wait_send()

    return pl.pallas_call(
        body,
        out_shape=jax.ShapeDtypeStruct((m, n), x.dtype),
        in_specs=[pl.BlockSpec(memory_space=pltpu.VMEM)],
        out_specs=pl.BlockSpec(memory_space=pltpu.VMEM),
        scratch_shapes=[
            pltpu.VMEM((m, n), x.dtype),
            pltpu.SemaphoreType.DMA((C,)),
            pltpu.SemaphoreType.DMA((C,)),
            pltpu.SemaphoreType.DMA((3, C)),
            pltpu.SemaphoreType.DMA((4, C)),
        ],
        compiler_params=pltpu.CompilerParams(collective_id=0),
    )(x)


# device time: 14963 ns/iter; 1.0903x vs baseline; 1.0903x over previous
import jax
import jax.numpy as jnp
from jax import lax
from jax.experimental import pallas as pl
from jax.experimental.pallas import tpu as pltpu

C = 4


def kernel(x):
    m, n = x.shape
    q = m // 4
    qc = q // C
    h = q // 2

    def body(x_ref, out_ref, graw_ref, zsend_sems, zrecv_sems,
             zdsend_sem, zdrecv_sem, f_send_sems, f_recv_sems):
        my_x = lax.axis_index("x")
        my_y = lax.axis_index("y")
        my_z = lax.axis_index("z")
        my_p = my_x * 2 + my_y

        zpartner = (my_x, my_y, 1 - my_z)
        xpeer = (1 - my_x, my_y, my_z)
        ypeer = (my_x, 1 - my_y, my_z)
        dpeer = (1 - my_x, 1 - my_y, my_z)
        px = (1 - my_x) * 2 + my_y
        py = my_x * 2 + (1 - my_y)
        pd = (1 - my_x) * 2 + (1 - my_y)

        barrier_sem = pltpu.get_barrier_semaphore()
        for peer in (zpartner, xpeer, ypeer, dpeer):
            pl.semaphore_signal(
                barrier_sem, inc=1,
                device_id=peer, device_id_type=pl.DeviceIdType.MESH,
            )
        pl.semaphore_wait(barrier_sem, 4)

        zrdmas = []
        for c in range(C):
            rows = pl.ds(my_p * q + c * qc, qc)
            r = pltpu.make_async_remote_copy(
                src_ref=x_ref.at[rows, :],
                dst_ref=graw_ref.at[rows, :],
                send_sem=zsend_sems.at[c],
                recv_sem=zrecv_sems.at[c],
                device_id=zpartner,
                device_id_type=pl.DeviceIdType.MESH,
            )
            r.start()
            zrdmas.append(r)
        drows = pl.ds(pd * q + h, h)
        zd = pltpu.make_async_remote_copy(
            src_ref=x_ref.at[drows, :],
            dst_ref=graw_ref.at[drows, :],
            send_sem=zdsend_sem,
            recv_sem=zdrecv_sem,
            device_id=zpartner,
            device_id_type=pl.DeviceIdType.MESH,
        )
        zd.start()

        fan = []
        for c in range(C):
            zrdmas[c].wait_recv()
            rows = pl.ds(my_p * q + c * qc, qc)
            peers = ((0, xpeer), (1, ypeer)) + (((2, dpeer),) if c < C // 2 else ())
            for i, peer in peers:
                r = pltpu.make_async_remote_copy(
                    src_ref=graw_ref.at[rows, :],
                    dst_ref=graw_ref.at[rows, :],
                    send_sem=f_send_sems.at[i, c],
                    recv_sem=f_recv_sems.at[my_p, c],
                    device_id=peer,
                    device_id_type=pl.DeviceIdType.MESH,
                )
                r.start()
                fan.append(r)

        myrows = pl.ds(my_p * q, q)
        out_ref[myrows, :] = x_ref[myrows, :] + graw_ref[myrows, :]

        def wait_chunk(sp, c):
            rows = pl.ds(sp * q + c * qc, qc)
            rr = pltpu.make_async_remote_copy(
                src_ref=graw_ref.at[rows, :],
                dst_ref=graw_ref.at[rows, :],
                send_sem=f_send_sems.at[0, 0],
                recv_sem=f_recv_sems.at[sp, c],
                device_id=xpeer,
                device_id_type=pl.DeviceIdType.MESH,
            )
            rr.wait_recv()

        for sp in (px, py):
            for c in range(C):
                wait_chunk(sp, c)
            qrows = pl.ds(sp * q, q)
            out_ref[qrows, :] = x_ref[qrows, :] + graw_ref[qrows, :]

        for c in range(C // 2):
            wait_chunk(pd, c)
        zdr = pltpu.make_async_remote_copy(
            src_ref=graw_ref.at[drows, :],
            dst_ref=graw_ref.at[drows, :],
            send_sem=zdsend_sem,
            recv_sem=zdrecv_sem,
            device_id=zpartner,
            device_id_type=pl.DeviceIdType.MESH,
        )
        zdr.wait_recv()
        qrows = pl.ds(pd * q, q)
        out_ref[qrows, :] = x_ref[qrows, :] + graw_ref[qrows, :]

        for r in zrdmas:
            r.wait_send()
        zd.wait_send()
        for r in fan:
            r.wait_send()

    return pl.pallas_call(
        body,
        out_shape=jax.ShapeDtypeStruct((m, n), x.dtype),
        in_specs=[pl.BlockSpec(memory_space=pltpu.VMEM)],
        out_specs=pl.BlockSpec(memory_space=pltpu.VMEM),
        scratch_shapes=[
            pltpu.VMEM((m, n), x.dtype),
            pltpu.SemaphoreType.DMA((C,)),
            pltpu.SemaphoreType.DMA((C,)),
            pltpu.SemaphoreType.DMA,
            pltpu.SemaphoreType.DMA,
            pltpu.SemaphoreType.DMA((3, C)),
            pltpu.SemaphoreType.DMA((4, C)),
        ],
        compiler_params=pltpu.CompilerParams(collective_id=0),
    )(x)
